# baseline (speedup 1.0000x reference)
import functools

import jax
import jax.numpy as jnp
import numpy as np
from jax import lax
from jax.experimental import pallas as pl
from jax.experimental.pallas import tpu as pltpu
from jax.experimental.pallas import tpu_sc as plsc

NC = 2
NS = 16
NW = NC * NS
CHUNK = 128
LW = 128


def _sc_gather4(user, item, t_mfu, t_mfi, t_mlu, t_mli):
    B = user.shape[0]
    bpw = B // NW
    nch = bpw // CHUNK
    mesh = plsc.VectorSubcoreMesh(core_axis_name="c", subcore_axis_name="s")
    out_sds = jax.ShapeDtypeStruct((B, LW), jnp.float32)

    @functools.partial(
        pl.kernel,
        mesh=mesh,
        out_type=[out_sds] * 4,
        scratch_types=[
            pltpu.VMEM((nch, CHUNK), jnp.int32),
            pltpu.VMEM((nch, CHUNK), jnp.int32),
            pltpu.VMEM((nch, CHUNK), jnp.int32),
            pltpu.VMEM((nch, CHUNK), jnp.int32),
            pltpu.VMEM((nch, CHUNK), jnp.int32),
            pltpu.VMEM((nch, CHUNK), jnp.int32),
            pltpu.VMEM((nch, CHUNK, LW), jnp.float32),
            pltpu.SemaphoreType.DMA,
            pltpu.SemaphoreType.DMA,
        ],
    )
    def k(user_hbm, item_hbm, h_mfu, h_mfi, h_mlu, h_mli,
          o_mfu, o_mfi, o_mlu, o_mli,
          uidx, iidx, u4, u2, i4, i2, ring, sem_g, sem_o):
        wid = lax.axis_index("s") * NC + lax.axis_index("c")
        base = wid * bpw
        for j in range(nch):
            pltpu.sync_copy(user_hbm.at[pl.ds(base + j * CHUNK, CHUNK)], uidx.at[j])
            pltpu.sync_copy(item_hbm.at[pl.ds(base + j * CHUNK, CHUNK)], iidx.at[j])
        for j in range(nch):
            for o in range(0, CHUNK, 16):
                s = pl.ds(o, 16)
                uv = uidx[j, s]
                iv = iidx[j, s]
                u4[j, s] = lax.shift_right_logical(uv, 4)
                u2[j, s] = lax.shift_right_logical(uv, 2)
                i4[j, s] = lax.shift_right_logical(iv, 4)
                i2[j, s] = lax.shift_right_logical(iv, 2)
        tables = ((h_mfu, u4, o_mfu), (h_mfi, i4, o_mfi),
                  (h_mlu, u2, o_mlu), (h_mli, i2, o_mli))
        outcopies = []
        for t, (tbl, ridx, out) in enumerate(tables):
            for c in outcopies:
                c.wait()
            gathers = [pltpu.async_copy(tbl.at[ridx.at[c]], ring.at[c], sem_g)
                       for c in range(nch)]
            for g in gathers:
                g.wait()
            outcopies = [
                pltpu.async_copy(ring.at[c],
                                 out.at[pl.ds(base + c * CHUNK, CHUNK)], sem_o)
                for c in range(nch)]
        for c in outcopies:
            c.wait()

    return k(user, item, t_mfu, t_mfi, t_mlu, t_mli)


def _tc_body(u_ref, it_ref, mfu_ref, mfi_ref, mlu_ref, mli_ref,
             w0u_ref, w0i_ref, b0_ref, w1_ref, b1_ref, w2_ref, b2_ref,
             s8_ref, wpm_ref, wph_ref, bp_ref, out_ref):
    f32 = jnp.float32
    R = mfu_ref.shape[0]
    col = lax.broadcasted_iota(jnp.int32, (R, LW), 1)
    u = u_ref[...]
    it = it_ref[...]
    zero = jnp.zeros((), f32)
    xu = jnp.where((col >> 5) == (u & 3), mlu_ref[...], zero)
    xi = jnp.where((col >> 5) == (it & 3), mli_ref[...], zero)
    h = jnp.dot(xu, w0u_ref[...], preferred_element_type=f32)
    h = h + jnp.dot(xi, w0i_ref[...], preferred_element_type=f32)
    h = jnp.maximum(h + b0_ref[...], 0.0)
    h = jnp.maximum(jnp.dot(h, w1_ref[...], preferred_element_type=f32) + b1_ref[...], 0.0)
    h = jnp.maximum(jnp.dot(h, w2_ref[...], preferred_element_type=f32) + b2_ref[...], 0.0)
    gu = jnp.where((col >> 3) == (u & 15), mfu_ref[...], zero)
    gi = jnp.where((col >> 3) == (it & 15), mfi_ref[...], zero)
    mfu_x = jnp.dot(gu, s8_ref[...], preferred_element_type=f32)
    mfi_x = jnp.dot(gi, s8_ref[...], preferred_element_type=f32)
    mf = mfu_x * mfi_x
    logit = (jnp.dot(mf, wpm_ref[...], preferred_element_type=f32)
             + jnp.dot(h, wph_ref[...], preferred_element_type=f32)
             + bp_ref[...])
    out_ref[...] = jax.nn.sigmoid(logit) * 4.5 + 0.5


def kernel(user, item, mf_user_emb, mf_item_emb, mlp_user_emb, mlp_item_emb,
           w0, b0, w1, b1, w2, b2, wp, bp):
    B = user.shape[0]
    dmf = mf_user_emb.shape[1]
    dml = mlp_user_emb.shape[1]
    nu = mf_user_emb.shape[0]
    ni = mf_item_emb.shape[0]

    t_mfu = mf_user_emb.reshape(nu * dmf // LW, LW)
    t_mfi = mf_item_emb.reshape(ni * dmf // LW, LW)
    t_mlu = mlp_user_emb.reshape(nu * dml // LW, LW)
    t_mli = mlp_item_emb.reshape(ni * dml // LW, LW)

    mfu, mfi, mlu, mli = _sc_gather4(user, item, t_mfu, t_mfi, t_mlu, t_mli)

    w0u = jnp.tile(w0[:, :dml].T, (LW // dml, 1))
    w0i = jnp.tile(w0[:, dml:].T, (LW // dml, 1))
    s8 = jnp.asarray(np.tile(np.eye(dmf, dtype=np.float32), (LW // dmf, 1)))
    w1t = w1.T
    w2t = w2.T
    wpm = wp[:, :dmf].T
    wph = wp[:, dmf:].T
    b0r = b0.reshape(1, -1)
    b1r = b1.reshape(1, -1)
    b2r = b2.reshape(1, -1)
    bpr = bp.reshape(1, 1)
    u2d = user.reshape(B, 1)
    i2d = item.reshape(B, 1)

    R = 2048
    d0 = w0.shape[0]
    d1 = w1.shape[0]
    d2 = w2.shape[0]
    data = lambda c: pl.BlockSpec((R, c), lambda i: (i, 0))
    full = lambda a, b: pl.BlockSpec((a, b), lambda i: (0, 0))
    out2 = pl.pallas_call(
        _tc_body,
        grid=(B // R,),
        in_specs=[
            data(1), data(1), data(LW), data(LW), data(LW), data(LW),
            full(LW, d0), full(LW, d0), full(1, d0),
            full(d0, d1), full(1, d1),
            full(d1, d2), full(1, d2),
            full(LW, dmf),
            full(dmf, 1), full(d2, 1), full(1, 1),
        ],
        out_specs=pl.BlockSpec((R, 1), lambda i: (i, 0)),
        out_shape=jax.ShapeDtypeStruct((B, 1), jnp.float32),
    )(u2d, i2d, mfu, mfi, mlu, mli, w0u, w0i, b0r, w1t, b1r, w2t, b2r,
      s8, wpm, wph, bpr)
    return out2.reshape(B)


# speedup vs baseline: 1.0006x; 1.0006x over previous
import functools

import jax
import jax.numpy as jnp
import numpy as np
from jax import lax
from jax.experimental import pallas as pl
from jax.experimental.pallas import tpu as pltpu
from jax.experimental.pallas import tpu_sc as plsc

NC = 2
NS = 16
NW = NC * NS
CHUNK = 128
LW = 128


def _sc_gather4(user, item, t_mfu, t_mfi, t_mlu, t_mli):
    B = user.shape[0]
    bpw = B // NW
    nch = bpw // CHUNK
    mesh = plsc.VectorSubcoreMesh(core_axis_name="c", subcore_axis_name="s")
    out_sds = jax.ShapeDtypeStruct((B, LW), jnp.float32)

    @functools.partial(
        pl.kernel,
        mesh=mesh,
        compiler_params=pltpu.CompilerParams(use_tc_tiling_on_sc=True),
        out_type=[out_sds] * 4,
        scratch_types=[
            pltpu.VMEM((nch, CHUNK), jnp.int32),
            pltpu.VMEM((nch, CHUNK), jnp.int32),
            pltpu.VMEM((nch, CHUNK), jnp.int32),
            pltpu.VMEM((nch, CHUNK), jnp.int32),
            pltpu.VMEM((nch, CHUNK), jnp.int32),
            pltpu.VMEM((nch, CHUNK), jnp.int32),
            pltpu.VMEM((nch, CHUNK, LW), jnp.float32),
            pltpu.SemaphoreType.DMA,
            pltpu.SemaphoreType.DMA,
        ],
    )
    def k(user_hbm, item_hbm, h_mfu, h_mfi, h_mlu, h_mli,
          o_mfu, o_mfi, o_mlu, o_mli,
          uidx, iidx, u4, u2, i4, i2, ring, sem_g, sem_o):
        wid = lax.axis_index("s") * NC + lax.axis_index("c")
        base = wid * bpw
        for j in range(nch):
            pltpu.sync_copy(user_hbm.at[pl.ds(base + j * CHUNK, CHUNK)], uidx.at[j])
            pltpu.sync_copy(item_hbm.at[pl.ds(base + j * CHUNK, CHUNK)], iidx.at[j])
        for j in range(nch):
            for o in range(0, CHUNK, 16):
                s = pl.ds(o, 16)
                uv = uidx[j, s]
                iv = iidx[j, s]
                u4[j, s] = lax.shift_right_logical(uv, 4)
                u2[j, s] = lax.shift_right_logical(uv, 2)
                i4[j, s] = lax.shift_right_logical(iv, 4)
                i2[j, s] = lax.shift_right_logical(iv, 2)
        tables = ((h_mfu, u4, o_mfu), (h_mfi, i4, o_mfi),
                  (h_mlu, u2, o_mlu), (h_mli, i2, o_mli))
        outcopies = []
        for t, (tbl, ridx, out) in enumerate(tables):
            for c in outcopies:
                c.wait()
            gathers = [pltpu.async_copy(tbl.at[ridx.at[c]], ring.at[c], sem_g)
                       for c in range(nch)]
            for g in gathers:
                g.wait()
            outcopies = [
                pltpu.async_copy(ring.at[c],
                                 out.at[pl.ds(base + c * CHUNK, CHUNK)], sem_o)
                for c in range(nch)]
        for c in outcopies:
            c.wait()

    return k(user, item, t_mfu, t_mfi, t_mlu, t_mli)


def _tc_body(u_ref, it_ref, mfu_ref, mfi_ref, mlu_ref, mli_ref,
             w0u_ref, w0i_ref, b0_ref, w1_ref, b1_ref, w2_ref, b2_ref,
             s8_ref, wpm_ref, wph_ref, bp_ref, out_ref):
    f32 = jnp.float32
    R = mfu_ref.shape[0]
    col = lax.broadcasted_iota(jnp.int32, (R, LW), 1)
    u = u_ref[...]
    it = it_ref[...]
    zero = jnp.zeros((), f32)
    xu = jnp.where((col >> 5) == (u & 3), mlu_ref[...], zero)
    xi = jnp.where((col >> 5) == (it & 3), mli_ref[...], zero)
    h = jnp.dot(xu, w0u_ref[...], preferred_element_type=f32)
    h = h + jnp.dot(xi, w0i_ref[...], preferred_element_type=f32)
    h = jnp.maximum(h + b0_ref[...], 0.0)
    h = jnp.maximum(jnp.dot(h, w1_ref[...], preferred_element_type=f32) + b1_ref[...], 0.0)
    h = jnp.maximum(jnp.dot(h, w2_ref[...], preferred_element_type=f32) + b2_ref[...], 0.0)
    gu = jnp.where((col >> 3) == (u & 15), mfu_ref[...], zero)
    gi = jnp.where((col >> 3) == (it & 15), mfi_ref[...], zero)
    mfu_x = jnp.dot(gu, s8_ref[...], preferred_element_type=f32)
    mfi_x = jnp.dot(gi, s8_ref[...], preferred_element_type=f32)
    mf = mfu_x * mfi_x
    logit = (jnp.dot(mf, wpm_ref[...], preferred_element_type=f32)
             + jnp.dot(h, wph_ref[...], preferred_element_type=f32)
             + bp_ref[...])
    out_ref[...] = jax.nn.sigmoid(logit) * 4.5 + 0.5


def kernel(user, item, mf_user_emb, mf_item_emb, mlp_user_emb, mlp_item_emb,
           w0, b0, w1, b1, w2, b2, wp, bp):
    B = user.shape[0]
    dmf = mf_user_emb.shape[1]
    dml = mlp_user_emb.shape[1]
    nu = mf_user_emb.shape[0]
    ni = mf_item_emb.shape[0]

    t_mfu = mf_user_emb.reshape(nu * dmf // LW, LW)
    t_mfi = mf_item_emb.reshape(ni * dmf // LW, LW)
    t_mlu = mlp_user_emb.reshape(nu * dml // LW, LW)
    t_mli = mlp_item_emb.reshape(ni * dml // LW, LW)

    mfu, mfi, mlu, mli = _sc_gather4(user, item, t_mfu, t_mfi, t_mlu, t_mli)

    w0u = jnp.tile(w0[:, :dml].T, (LW // dml, 1))
    w0i = jnp.tile(w0[:, dml:].T, (LW // dml, 1))
    s8 = jnp.asarray(np.tile(np.eye(dmf, dtype=np.float32), (LW // dmf, 1)))
    w1t = w1.T
    w2t = w2.T
    wpm = wp[:, :dmf].T
    wph = wp[:, dmf:].T
    b0r = b0.reshape(1, -1)
    b1r = b1.reshape(1, -1)
    b2r = b2.reshape(1, -1)
    bpr = bp.reshape(1, 1)
    u2d = user.reshape(B, 1)
    i2d = item.reshape(B, 1)

    R = 2048
    d0 = w0.shape[0]
    d1 = w1.shape[0]
    d2 = w2.shape[0]
    data = lambda c: pl.BlockSpec((R, c), lambda i: (i, 0))
    full = lambda a, b: pl.BlockSpec((a, b), lambda i: (0, 0))
    out2 = pl.pallas_call(
        _tc_body,
        grid=(B // R,),
        in_specs=[
            data(1), data(1), data(LW), data(LW), data(LW), data(LW),
            full(LW, d0), full(LW, d0), full(1, d0),
            full(d0, d1), full(1, d1),
            full(d1, d2), full(1, d2),
            full(LW, dmf),
            full(dmf, 1), full(d2, 1), full(1, 1),
        ],
        out_specs=pl.BlockSpec((R, 1), lambda i: (i, 0)),
        out_shape=jax.ShapeDtypeStruct((B, 1), jnp.float32),
    )(u2d, i2d, mfu, mfi, mlu, mli, w0u, w0i, b0r, w1t, b1r, w2t, b2r,
      s8, wpm, wph, bpr)
    return out2.reshape(B)
